# Initial kernel scaffold; baseline (speedup 1.0000x reference)
import jax
import jax.numpy as jnp
from jax.experimental import pallas as pl


def kernel(x, Wq, Wk, Wv, Wo, positions, log_scales, amplitudes):
    raise NotImplementedError("write your pallas kernel here")


# speedup vs baseline: 1.0874x; 1.0874x over previous
import functools

import jax
import jax.numpy as jnp
from jax.experimental import pallas as pl
from jax.experimental.pallas import tpu as pltpu

_SB = 512


def _f32dot(a, b):
    return jax.lax.dot_general(a, b, (((1,), (0,)), ((), ())),
                               preferred_element_type=jnp.float32)


def _pass1_body(x_ref, wk_ref, wv_ref, pbd_ref, m_ref, psq_ref, itv_ref,
                ss_ref, norm_ref):
    c = pl.program_id(1)
    xb = x_ref[0]
    k = _f32dot(xb, wk_ref[...])
    v = _f32dot(xb, wv_ref[...])
    kp = _f32dot(k, pbd_ref[...])
    k2s = _f32dot(k * k, m_ref[...])
    dk = k2s + psq_ref[...] - 2.0 * kp
    ak = jnp.exp(-jnp.maximum(dk, 0.0) * itv_ref[...])
    ssc = jax.lax.dot_general(ak, v, (((0,), (0,)), ((), ())),
                              preferred_element_type=jnp.float32)
    nc = jnp.sum(ak, axis=0, keepdims=True)

    @pl.when(c == 0)
    def _():
        ss_ref[0] = ssc
        norm_ref[0] = nc

    @pl.when(c != 0)
    def _():
        ss_ref[0] += ssc
        norm_ref[0] += nc

    @pl.when(c == pl.num_programs(1) - 1)
    def _():
        ss_ref[0] = ss_ref[0] * m_ref[...]


def _pass2_body(x_ref, wq_ref, pbd_ref, m_ref, psq_ref, itv_ref, amp_ref,
                wo_ref, ss_ref, norm_ref, out_ref):
    xb = x_ref[0]
    q = _f32dot(xb, wq_ref[...])
    qp = _f32dot(q, pbd_ref[...])
    q2s = _f32dot(q * q, m_ref[...])
    dq = q2s + psq_ref[...] - 2.0 * qp
    w = jnp.exp(-jnp.maximum(dq, 0.0) * itv_ref[...]) * amp_ref[...]
    num = _f32dot(w, ss_ref[0])
    den = _f32dot(w * norm_ref[0], m_ref[...]) + 1e-8
    y = num / den
    out_ref[0] = _f32dot(y, wo_ref[...])


def kernel(x, Wq, Wk, Wv, Wo, positions, log_scales, amplitudes):
    B, S, D = x.shape
    H, K, DH = positions.shape
    f32 = jnp.float32

    scales = jnp.exp(log_scales)
    itv = (0.5 / (scales * scales + 1e-6)).reshape(1, H * K)
    psq = jnp.sum(positions * positions, axis=-1).reshape(1, H * K)
    amp = amplitudes.reshape(1, H * K)
    eye_h = jnp.eye(H, dtype=f32)
    pbd = jnp.einsum('hg,hkd->hdgk', eye_h, positions).reshape(D, D)
    m = jnp.kron(eye_h, jnp.ones((DH, K), f32))

    nc = S // _SB
    grid = (B, nc)

    full = lambda b, c: (0, 0)
    xspec = pl.BlockSpec((1, _SB, D), lambda b, c: (b, c, 0))
    wspec = pl.BlockSpec((D, D), full)
    vspec = pl.BlockSpec((1, D), full)
    ss_spec = pl.BlockSpec((1, D, D), lambda b, c: (b, 0, 0))
    nm_spec = pl.BlockSpec((1, 1, D), lambda b, c: (b, 0, 0))

    ss, norm = pl.pallas_call(
        _pass1_body,
        grid=grid,
        in_specs=[xspec, wspec, wspec, wspec, wspec, vspec, vspec],
        out_specs=[ss_spec, nm_spec],
        out_shape=[jax.ShapeDtypeStruct((B, D, D), f32),
                   jax.ShapeDtypeStruct((B, 1, D), f32)],
        compiler_params=pltpu.CompilerParams(
            dimension_semantics=("arbitrary", "arbitrary")),
    )(x, Wk, Wv, pbd, m, psq, itv)

    out = pl.pallas_call(
        _pass2_body,
        grid=grid,
        in_specs=[xspec, wspec, wspec, wspec, vspec, vspec, vspec, wspec,
                  ss_spec, nm_spec],
        out_specs=xspec,
        out_shape=jax.ShapeDtypeStruct((B, S, D), f32),
        compiler_params=pltpu.CompilerParams(
            dimension_semantics=("parallel", "arbitrary")),
    )(x, Wq, pbd, m, psq, itv, amp, Wo, ss, norm)
    return out
